# baseline (speedup 1.0000x reference)
import functools
import math

import jax
import jax.numpy as jnp
from jax import lax
from jax.experimental import pallas as pl
from jax.experimental.pallas import tpu as pltpu
from jax.experimental.pallas import tpu_sc as plsc

D_MODEL = 64
SCALE = math.sqrt(D_MODEL)
NC, NS, L = 2, 16, 16
NW = NC * NS
CH = 128
NBUF = 4


def _tec_body(nch, x_hbm, table_hbm, out_hbm, idx_v, gbuf, obuf, gsem, osem):
    wid = lax.axis_index("c") * NS + lax.axis_index("s")

    pltpu.sync_copy(x_hbm.at[wid], idx_v)

    def gather(g, b):
        return pltpu.make_async_copy(
            table_hbm.at[idx_v.at[g]], gbuf.at[b], gsem.at[b])

    def put(g, b):
        return pltpu.make_async_copy(
            obuf.at[b], out_hbm.at[wid, g], osem.at[b])

    for b in range(NBUF):
        gather(b, b).start()

    niter = nch // NBUF

    @pl.loop(0, niter)
    def chunk_loop(g0):
        for b in range(NBUF):
            g = g0 * NBUF + b
            gather(g, b).wait()

            @pl.when(g0 > 0)
            def _wait_prev_put():
                put(g - NBUF, b).wait()

            @pl.loop(0, CH, unroll=8)
            def _scale(r):
                for c in range(D_MODEL // L):
                    sl = pl.ds(c * L, L)
                    obuf[b, r, sl] = gbuf[b, r, sl] * SCALE

            put(g, b).start()

            @pl.when(g0 < niter - 1)
            def _next_gather():
                gather(g + NBUF, b).start()

    for b in range(NBUF):
        put((niter - 1) * NBUF + b, b).wait()


def kernel(x, table):
    b_dim, s_dim = x.shape
    total = b_dim * s_dim
    nch = total // (NW * CH)
    assert nch * NW * CH == total and nch % NBUF == 0

    x_r = x.reshape(NW, nch, CH).astype(jnp.int32)

    mesh = plsc.VectorSubcoreMesh(
        core_axis_name="c", subcore_axis_name="s",
        num_cores=NC, num_subcores=NS)

    sc_call = pl.kernel(
        functools.partial(_tec_body, nch),
        out_type=jax.ShapeDtypeStruct((NW, nch, CH, D_MODEL), jnp.float32),
        mesh=mesh,
        scratch_types=[
            pltpu.VMEM((nch, CH), jnp.int32),
            pltpu.VMEM((NBUF, CH, D_MODEL), jnp.float32),
            pltpu.VMEM((NBUF, CH, D_MODEL), jnp.float32),
            pltpu.SemaphoreType.DMA((NBUF,)),
            pltpu.SemaphoreType.DMA((NBUF,)),
        ],
        compiler_params=pltpu.CompilerParams(use_tc_tiling_on_sc=False),
    )
    out = sc_call(x_r, table)
    return out.reshape(b_dim, s_dim, D_MODEL)


# speedup vs baseline: 1.1030x; 1.1030x over previous
import functools
import math

import jax
import jax.numpy as jnp
from jax import lax
from jax.experimental import pallas as pl
from jax.experimental.pallas import tpu as pltpu
from jax.experimental.pallas import tpu_sc as plsc

D_MODEL = 64
SCALE = math.sqrt(D_MODEL)
NC, NS, L = 2, 16, 16
NW = NC * NS
CH = 128
NBUF = 4


def _tec_body(nch, x_hbm, table_hbm, out_hbm, idx_v, gbuf, obuf, gsem, osem):
    wid = lax.axis_index("c") * NS + lax.axis_index("s")

    pltpu.sync_copy(x_hbm.at[wid], idx_v)

    def gather(g, b):
        return pltpu.make_async_copy(
            table_hbm.at[idx_v.at[g]], gbuf.at[b], gsem.at[b])

    def put(g, b):
        base2 = (wid * nch * CH + g * CH) // 2
        return pltpu.make_async_copy(
            obuf.at[b], out_hbm.at[pl.ds(base2, CH // 2)], osem.at[b])

    for b in range(NBUF):
        gather(b, b).start()

    niter = nch // NBUF

    @pl.loop(0, niter)
    def chunk_loop(g0):
        for b in range(NBUF):
            g = g0 * NBUF + b
            gather(g, b).wait()

            @pl.when(g0 > 0)
            def _wait_prev_put():
                put(g - NBUF, b).wait()

            @plsc.parallel_loop(0, CH // 2, unroll=4)
            def _scale(r2):
                nv = 2 * D_MODEL // L
                vals = [
                    gbuf[b, 2 * r2 + (c * L) // D_MODEL,
                         pl.ds((c * L) % D_MODEL, L)]
                    for c in range(nv)
                ]
                for c in range(nv):
                    obuf[b, r2, pl.ds(c * L, L)] = vals[c] * SCALE

            put(g, b).start()

            @pl.when(g0 < niter - 1)
            def _next_gather():
                gather(g + NBUF, b).start()

    for b in range(NBUF):
        put((niter - 1) * NBUF + b, b).wait()


def kernel(x, table):
    b_dim, s_dim = x.shape
    total = b_dim * s_dim
    nch = total // (NW * CH)
    assert nch * NW * CH == total and nch % NBUF == 0

    x_r = x.reshape(NW, nch, CH).astype(jnp.int32)

    mesh = plsc.VectorSubcoreMesh(
        core_axis_name="c", subcore_axis_name="s",
        num_cores=NC, num_subcores=NS)

    sc_call = pl.kernel(
        functools.partial(_tec_body, nch),
        out_type=jax.ShapeDtypeStruct((total // 2, 2 * D_MODEL), jnp.float32),
        mesh=mesh,
        scratch_types=[
            pltpu.VMEM((nch, CH), jnp.int32),
            pltpu.VMEM((NBUF, CH, D_MODEL), jnp.float32),
            pltpu.VMEM((NBUF, CH // 2, 2 * D_MODEL), jnp.float32),
            pltpu.SemaphoreType.DMA((NBUF,)),
            pltpu.SemaphoreType.DMA((NBUF,)),
        ],
        compiler_params=pltpu.CompilerParams(use_tc_tiling_on_sc=False),
    )
    out = sc_call(x_r, table)
    return out.reshape(b_dim, s_dim, D_MODEL)
